# initial kernel scaffold (unmeasured)
import functools

import jax
import jax.numpy as jnp
from jax import lax
from jax.experimental import pallas as pl
from jax.experimental.pallas import tpu as pltpu

X, Y, Z = 2, 2, 4
M = 2048
KS = 8192 // (X * Z)
HALF, QUART, CHUNK = M // 2, M // 4, M // 16

RSX, RSY = 0, 1
RSZ = 2
AGZ = 5
AGY, AGX = 8, 9
NSEM = 10


def kernel(dy, W):
    m, _ = dy.shape
    xi = lax.axis_index("x")
    zi = lax.axis_index("z")
    idx = xi * Z + zi
    dy_c = lax.dynamic_slice(dy, (0, idx * KS), (m, KS))
    w_c = lax.dynamic_slice(W, (0, idx * KS), (m, KS))

    def body(dy_ref, w_ref, out_ref, comm_x, comm_y, comm_z, ssem, rsem, credit):
        x = lax.axis_index("x")
        y = lax.axis_index("y")
        z = lax.axis_index("z")
        xp = (1 - x, y, z)
        yp = (x, 1 - y, z)
        zl = (x, y, (z - 1) % Z)
        zr = (x, y, (z + 1) % Z)

        out_ref[...] = lax.dot_general(
            dy_ref[...],
            w_ref[...],
            dimension_numbers=(((1,), (1,)), ((), ())),
            preferred_element_type=jnp.float32,
        )

        barrier = pltpu.get_barrier_semaphore()
        for nbr in (xp, yp, zl, zr):
            pl.semaphore_signal(
                barrier, inc=1, device_id=nbr,
                device_id_type=pl.DeviceIdType.MESH,
            )
        pl.semaphore_wait(barrier, 4)

        my_half = x * HALF
        oth_half = (1 - x) * HALF
        my_q = my_half + y * QUART
        oth_q = my_half + (1 - y) * QUART

        def copy(src, dst, sem_i, dev):
            return pltpu.make_async_remote_copy(
                src_ref=src, dst_ref=dst,
                send_sem=ssem.at[sem_i], recv_sem=rsem.at[sem_i],
                device_id=dev, device_id_type=pl.DeviceIdType.MESH,
            )

        rdma = copy(out_ref.at[pl.ds(oth_half, HALF), :], comm_x, RSX, xp)
        rdma.start()
        rdma.wait()
        out_ref[pl.ds(my_half, HALF), :] = (
            out_ref[pl.ds(my_half, HALF), :] + comm_x[...]
        )

        rdma = copy(out_ref.at[pl.ds(oth_q, QUART), :], comm_y, RSY, yp)
        rdma.start()
        rdma.wait()
        out_ref[pl.ds(my_q, QUART), :] = (
            out_ref[pl.ds(my_q, QUART), :] + comm_y[...]
        )

        for s in range(3):
            send_c = (z - s) % Z
            recv_c = (z - 1 - s) % Z
            slot = s % 2
            if s == 2:
                pl.semaphore_wait(credit, 1)
            rdma = copy(
                out_ref.at[pl.ds(my_q + send_c * CHUNK, CHUNK), :],
                comm_z.at[slot], RSZ + s, zr,
            )
            rdma.start()
            rdma.wait()
            out_ref[pl.ds(my_q + recv_c * CHUNK, CHUNK), :] = (
                out_ref[pl.ds(my_q + recv_c * CHUNK, CHUNK), :]
                + comm_z[slot, :, :]
            )
            if s == 0:
                pl.semaphore_signal(
                    credit, inc=1, device_id=zl,
                    device_id_type=pl.DeviceIdType.MESH,
                )

        for s in range(3):
            send_c = (z + 1 - s) % Z
            rdma = copy(
                out_ref.at[pl.ds(my_q + send_c * CHUNK, CHUNK), :],
                out_ref.at[pl.ds(my_q + send_c * CHUNK, CHUNK), :],
                AGZ + s, zr,
            )
            rdma.start()
            rdma.wait()

        rdma = copy(
            out_ref.at[pl.ds(my_q, QUART), :],
            out_ref.at[pl.ds(my_q, QUART), :],
            AGY, yp,
        )
        rdma.start()
        rdma.wait()

        rdma = copy(
            out_ref.at[pl.ds(my_half, HALF), :],
            out_ref.at[pl.ds(my_half, HALF), :],
            AGX, xp,
        )
        rdma.start()
        rdma.wait()

        @functools.partial(pl.run_scoped, sem2=pltpu.SemaphoreType.REGULAR)
        def _(sem2):
            for nbr in (xp, yp, zl, zr):
                pl.semaphore_signal(
                    sem2, inc=1, device_id=nbr,
                    device_id_type=pl.DeviceIdType.MESH,
                )
            pl.semaphore_wait(sem2, 4)

    return pl.pallas_call(
        body,
        out_shape=jax.ShapeDtypeStruct((M, M), jnp.float32),
        in_specs=[
            pl.BlockSpec(memory_space=pltpu.VMEM),
            pl.BlockSpec(memory_space=pltpu.VMEM),
        ],
        out_specs=pl.BlockSpec(memory_space=pltpu.VMEM),
        scratch_shapes=[
            pltpu.VMEM((HALF, M), jnp.float32),
            pltpu.VMEM((QUART, M), jnp.float32),
            pltpu.VMEM((2, CHUNK, M), jnp.float32),
            pltpu.SemaphoreType.DMA((NSEM,)),
            pltpu.SemaphoreType.DMA((NSEM,)),
            pltpu.SemaphoreType.REGULAR,
        ],
        compiler_params=pltpu.CompilerParams(collective_id=0),
    )(dy_c, w_c)


# baseline (device time: 410906 ns/iter reference)
import functools

import jax
import jax.numpy as jnp
from jax import lax
from jax.experimental import pallas as pl
from jax.experimental.pallas import tpu as pltpu

X, Y, Z = 2, 2, 4
M = 2048
KS = 8192 // (X * Z)
HALF, QUART, CHUNK = M // 2, M // 4, M // 16

RSX, RSY = 0, 1
RSZ = 2
AGZ = 5
AGY, AGX = 8, 9
NSEM = 10


def kernel(dy, W):
    m, _ = dy.shape
    xi = lax.axis_index("x")
    zi = lax.axis_index("z")
    idx = xi * Z + zi
    dy_c = lax.dynamic_slice(dy, (0, idx * KS), (m, KS))
    w_c = lax.dynamic_slice(W, (0, idx * KS), (m, KS))

    def body(dy_ref, w_ref, out_ref, comm_x, comm_y, comm_z, ssem, rsem, credit):
        x = lax.axis_index("x")
        y = lax.axis_index("y")
        z = lax.axis_index("z")
        xp = (1 - x, y, z)
        yp = (x, 1 - y, z)
        zl = (x, y, (z - 1) % Z)
        zr = (x, y, (z + 1) % Z)

        out_ref[...] = lax.dot_general(
            dy_ref[...],
            w_ref[...],
            dimension_numbers=(((1,), (1,)), ((), ())),
            preferred_element_type=jnp.float32,
        )

        barrier = pltpu.get_barrier_semaphore()
        for nbr in (xp, yp, zl, zr):
            pl.semaphore_signal(
                barrier, inc=1, device_id=nbr,
                device_id_type=pl.DeviceIdType.MESH,
            )
        pl.semaphore_wait(barrier, 4)

        my_half = x * HALF
        oth_half = (1 - x) * HALF
        my_q = my_half + y * QUART
        oth_q = my_half + (1 - y) * QUART

        def copy(src, dst, sem_i, dev):
            return pltpu.make_async_remote_copy(
                src_ref=src, dst_ref=dst,
                send_sem=ssem.at[sem_i], recv_sem=rsem.at[sem_i],
                device_id=dev, device_id_type=pl.DeviceIdType.MESH,
            )

        rdma = copy(out_ref.at[pl.ds(oth_half, HALF), :], comm_x, RSX, xp)
        rdma.start()
        rdma.wait()
        out_ref[pl.ds(my_half, HALF), :] = (
            out_ref[pl.ds(my_half, HALF), :] + comm_x[...]
        )

        rdma = copy(out_ref.at[pl.ds(oth_q, QUART), :], comm_y, RSY, yp)
        rdma.start()
        rdma.wait()
        out_ref[pl.ds(my_q, QUART), :] = (
            out_ref[pl.ds(my_q, QUART), :] + comm_y[...]
        )

        for s in range(3):
            send_c = (z - s) % Z
            recv_c = (z - 1 - s) % Z
            slot = s % 2
            if s == 2:
                pl.semaphore_wait(credit, 1)
            rdma = copy(
                out_ref.at[pl.ds(my_q + send_c * CHUNK, CHUNK), :],
                comm_z.at[slot], RSZ + s, zr,
            )
            rdma.start()
            rdma.wait()
            out_ref[pl.ds(my_q + recv_c * CHUNK, CHUNK), :] = (
                out_ref[pl.ds(my_q + recv_c * CHUNK, CHUNK), :]
                + comm_z[slot, :, :]
            )
            if s == 0:
                pl.semaphore_signal(
                    credit, inc=1, device_id=zl,
                    device_id_type=pl.DeviceIdType.MESH,
                )

        for s in range(3):
            send_c = (z + 1 - s) % Z
            rdma = copy(
                out_ref.at[pl.ds(my_q + send_c * CHUNK, CHUNK), :],
                out_ref.at[pl.ds(my_q + send_c * CHUNK, CHUNK), :],
                AGZ + s, zr,
            )
            rdma.start()
            rdma.wait()

        rdma = copy(
            out_ref.at[pl.ds(my_q, QUART), :],
            out_ref.at[pl.ds(my_q, QUART), :],
            AGY, yp,
        )
        rdma.start()
        rdma.wait()

        rdma = copy(
            out_ref.at[pl.ds(my_half, HALF), :],
            out_ref.at[pl.ds(my_half, HALF), :],
            AGX, xp,
        )
        rdma.start()
        rdma.wait()

        @functools.partial(pl.run_scoped, sem2=pltpu.SemaphoreType.REGULAR)
        def _(sem2):
            for nbr in (xp, yp, zl, zr):
                pl.semaphore_signal(
                    sem2, inc=1, device_id=nbr,
                    device_id_type=pl.DeviceIdType.MESH,
                )
            pl.semaphore_wait(sem2, 4)

    return pl.pallas_call(
        body,
        out_shape=jax.ShapeDtypeStruct((M, M), jnp.float32),
        in_specs=[
            pl.BlockSpec(memory_space=pltpu.VMEM),
            pl.BlockSpec(memory_space=pltpu.VMEM),
        ],
        out_specs=pl.BlockSpec(memory_space=pltpu.VMEM),
        scratch_shapes=[
            pltpu.VMEM((HALF, M), jnp.float32),
            pltpu.VMEM((QUART, M), jnp.float32),
            pltpu.VMEM((2, CHUNK, M), jnp.float32),
            pltpu.SemaphoreType.DMA((NSEM,)),
            pltpu.SemaphoreType.DMA((NSEM,)),
            pltpu.SemaphoreType.REGULAR,
        ],
        compiler_params=pltpu.CompilerParams(
            collective_id=0,
            vmem_limit_bytes=60 * 1024 * 1024,
        ),
    )(dy_c, w_c)


# device time: 272339 ns/iter; 1.5088x vs baseline; 1.5088x over previous
import functools

import jax
import jax.numpy as jnp
from jax import lax
from jax.experimental import pallas as pl
from jax.experimental.pallas import tpu as pltpu

X, Y, Z = 2, 2, 4
M = 2048
KS = 8192 // (X * Z)
M2 = M // 2
H2 = M // 4
Q2 = M // 8
C2 = M // 32

A_RSX, A_RSY, A_RSZ, A_AGZ, A_AGY, A_AGX = 0, 1, 2, 5, 8, 9
B_RSY, B_RSX, B_RSZ, B_AGZ, B_AGX, B_AGY = 10, 11, 12, 15, 18, 19
NSEM = 20

MESH = pl.DeviceIdType.MESH


def kernel(dy, W):
    m, _ = dy.shape
    xi = lax.axis_index("x")
    zi = lax.axis_index("z")
    idx = xi * Z + zi
    dy_c = lax.dynamic_slice(dy, (0, idx * KS), (m, KS))
    w_c = lax.dynamic_slice(W, (0, idx * KS), (m, KS))

    def body(dy_ref, w_ref, out_ref, c_ax, c_ay, c_az, c_by, c_bx, c_bz,
             ssem, rsem, cred_a, cred_b):
        x = lax.axis_index("x")
        y = lax.axis_index("y")
        z = lax.axis_index("z")
        xp = (1 - x, y, z)
        yp = (x, 1 - y, z)
        zl = (x, y, (z - 1) % Z)
        zr = (x, y, (z + 1) % Z)

        barrier = pltpu.get_barrier_semaphore()
        for nbr in (xp, yp, zl, zr):
            pl.semaphore_signal(barrier, inc=1, device_id=nbr,
                                device_id_type=MESH)
        pl.semaphore_wait(barrier, 4)

        a_my, a_oth = x * H2, (1 - x) * H2
        b_my, b_oth = M2 + y * H2, M2 + (1 - y) * H2
        a_myq, a_othq = a_my + y * Q2, a_my + (1 - y) * Q2
        b_myq, b_othq = b_my + x * Q2, b_my + (1 - x) * Q2

        def mm(r0, nrows):
            out_ref[pl.ds(r0, nrows), :] = lax.dot_general(
                dy_ref[pl.ds(r0, nrows), :], w_ref[...],
                dimension_numbers=(((1,), (1,)), ((), ())),
                preferred_element_type=jnp.float32,
            )

        def copy(src, dst, sem_i, dev):
            return pltpu.make_async_remote_copy(
                src_ref=src, dst_ref=dst,
                send_sem=ssem.at[sem_i], recv_sem=rsem.at[sem_i],
                device_id=dev, device_id_type=MESH,
            )

        def add(r0, nrows, buf):
            out_ref[pl.ds(r0, nrows), :] = (
                out_ref[pl.ds(r0, nrows), :] + buf
            )

        mm(a_oth, H2)
        mm(b_oth, H2)

        ra = copy(out_ref.at[pl.ds(a_oth, H2), :], c_ax, A_RSX, xp)
        rb = copy(out_ref.at[pl.ds(b_oth, H2), :], c_by, B_RSY, yp)
        ra.start()
        rb.start()
        mm(a_my, H2)
        mm(b_my, H2)
        ra.wait()
        add(a_my, H2, c_ax[...])
        rb.wait()
        add(b_my, H2, c_by[...])

        ra = copy(out_ref.at[pl.ds(a_othq, Q2), :], c_ay, A_RSY, yp)
        rb = copy(out_ref.at[pl.ds(b_othq, Q2), :], c_bx, B_RSX, xp)
        ra.start()
        rb.start()
        ra.wait()
        add(a_myq, Q2, c_ay[...])
        rb.wait()
        add(b_myq, Q2, c_bx[...])

        for s in range(3):
            slot = s % 2
            a_sc, a_rc = (z - s) % Z, (z - 1 - s) % Z
            b_sc, b_rc = (z + s) % Z, (z + 1 + s) % Z
            if s == 2:
                pl.semaphore_wait(cred_a, 1)
                pl.semaphore_wait(cred_b, 1)
            ra = copy(out_ref.at[pl.ds(a_myq + a_sc * C2, C2), :],
                      c_az.at[slot], A_RSZ + s, zr)
            rb = copy(out_ref.at[pl.ds(b_myq + b_sc * C2, C2), :],
                      c_bz.at[slot], B_RSZ + s, zl)
            ra.start()
            rb.start()
            ra.wait()
            add(a_myq + a_rc * C2, C2, c_az[slot, :, :])
            rb.wait()
            add(b_myq + b_rc * C2, C2, c_bz[slot, :, :])
            if s == 0:
                pl.semaphore_signal(cred_a, inc=1, device_id=zl,
                                    device_id_type=MESH)
                pl.semaphore_signal(cred_b, inc=1, device_id=zr,
                                    device_id_type=MESH)

        for s in range(3):
            a_c = (z + 1 - s) % Z
            b_c = (z - 1 + s) % Z
            ra = copy(out_ref.at[pl.ds(a_myq + a_c * C2, C2), :],
                      out_ref.at[pl.ds(a_myq + a_c * C2, C2), :],
                      A_AGZ + s, zr)
            rb = copy(out_ref.at[pl.ds(b_myq + b_c * C2, C2), :],
                      out_ref.at[pl.ds(b_myq + b_c * C2, C2), :],
                      B_AGZ + s, zl)
            ra.start()
            rb.start()
            ra.wait()
            rb.wait()

        ra = copy(out_ref.at[pl.ds(a_myq, Q2), :],
                  out_ref.at[pl.ds(a_myq, Q2), :], A_AGY, yp)
        rb = copy(out_ref.at[pl.ds(b_myq, Q2), :],
                  out_ref.at[pl.ds(b_myq, Q2), :], B_AGX, xp)
        ra.start()
        rb.start()
        ra.wait()
        rb.wait()

        ra = copy(out_ref.at[pl.ds(a_my, H2), :],
                  out_ref.at[pl.ds(a_my, H2), :], A_AGX, xp)
        rb = copy(out_ref.at[pl.ds(b_my, H2), :],
                  out_ref.at[pl.ds(b_my, H2), :], B_AGY, yp)
        ra.start()
        rb.start()
        ra.wait()
        rb.wait()

        @functools.partial(pl.run_scoped, sem2=pltpu.SemaphoreType.REGULAR)
        def _(sem2):
            for nbr in (xp, yp, zl, zr):
                pl.semaphore_signal(sem2, inc=1, device_id=nbr,
                                    device_id_type=MESH)
            pl.semaphore_wait(sem2, 4)

    return pl.pallas_call(
        body,
        out_shape=jax.ShapeDtypeStruct((M, M), jnp.float32),
        in_specs=[
            pl.BlockSpec(memory_space=pltpu.VMEM),
            pl.BlockSpec(memory_space=pltpu.VMEM),
        ],
        out_specs=pl.BlockSpec(memory_space=pltpu.VMEM),
        scratch_shapes=[
            pltpu.VMEM((H2, M), jnp.float32),
            pltpu.VMEM((Q2, M), jnp.float32),
            pltpu.VMEM((2, C2, M), jnp.float32),
            pltpu.VMEM((H2, M), jnp.float32),
            pltpu.VMEM((Q2, M), jnp.float32),
            pltpu.VMEM((2, C2, M), jnp.float32),
            pltpu.SemaphoreType.DMA((NSEM,)),
            pltpu.SemaphoreType.DMA((NSEM,)),
            pltpu.SemaphoreType.REGULAR,
            pltpu.SemaphoreType.REGULAR,
        ],
        compiler_params=pltpu.CompilerParams(
            collective_id=0,
            vmem_limit_bytes=60 * 1024 * 1024,
        ),
    )(dy_c, w_c)


# device time: 253500 ns/iter; 1.6209x vs baseline; 1.0743x over previous
import functools

import jax
import jax.numpy as jnp
from jax import lax
from jax.experimental import pallas as pl
from jax.experimental.pallas import tpu as pltpu

X, Y, Z = 2, 2, 4
M = 2048
KS = 8192 // (X * Z)
S3 = 512
H3 = 256
Q3 = 128
C3 = 32

P1S, P2S, RSZS, AGZS, AGQS, AGHS = 0, 1, 2, 5, 8, 9
NSEM = 40

MESH = pl.DeviceIdType.MESH


def kernel(dy, W):
    m, _ = dy.shape
    xi = lax.axis_index("x")
    zi = lax.axis_index("z")
    idx = xi * Z + zi
    dy_c = lax.dynamic_slice(dy, (0, idx * KS), (m, KS))
    w_c = lax.dynamic_slice(W, (0, idx * KS), (m, KS))

    def body(dy_ref, w_ref, out_ref, CX, CY, CZ, ssem, rsem,
             cred0, cred1, cred2, cred3):
        x = lax.axis_index("x")
        y = lax.axis_index("y")
        z = lax.axis_index("z")
        xp = (1 - x, y, z)
        yp = (x, 1 - y, z)
        zl = (x, y, (z - 1) % Z)
        zr = (x, y, (z + 1) % Z)
        creds = [cred0, cred1, cred2, cred3]

        def mm(r0, nrows):
            out_ref[pl.ds(r0, nrows), :] = lax.dot_general(
                dy_ref[pl.ds(r0, nrows), :], w_ref[...],
                dimension_numbers=(((1,), (1,)), ((), ())),
                preferred_element_type=jnp.float32,
            )

        def copy(src, dst, sem_i, dev):
            return pltpu.make_async_remote_copy(
                src_ref=src, dst_ref=dst,
                send_sem=ssem.at[sem_i], recv_sem=rsem.at[sem_i],
                device_id=dev, device_id_type=MESH,
            )

        def add(r0, nrows, buf):
            out_ref[pl.ds(r0, nrows), :] = (
                out_ref[pl.ds(r0, nrows), :] + buf
            )

        class Chain:
            def __init__(self, i, base, xfirst, d):
                self.i, self.d, self.sb = i, d, 10 * i
                self.cred = creds[i]
                if xfirst:
                    self.f_my, self.f_oth = base + x * H3, base + (1 - x) * H3
                    self.q_my = self.f_my + y * Q3
                    self.q_oth = self.f_my + (1 - y) * Q3
                    self.p1p, self.p2p = xp, yp
                else:
                    self.f_my, self.f_oth = base + y * H3, base + (1 - y) * H3
                    self.q_my = self.f_my + x * Q3
                    self.q_oth = self.f_my + (1 - x) * Q3
                    self.p1p, self.p2p = yp, xp
                self.snbr = zr if d == 1 else zl
                self.rnbr = zl if d == 1 else zr
                self.r = None

            def rs_send(self, s):
                return (z - self.d * s) % Z

            def rs_recv(self, s):
                return (z - self.d * (s + 1)) % Z

            def ag_send(self, s):
                return (z + self.d * (1 - s)) % Z

            def p1_start(self):
                self.r = copy(out_ref.at[pl.ds(self.f_oth, H3), :],
                              CX.at[self.i], self.sb + P1S, self.p1p)
                self.r.start()

            def p1_fin(self):
                self.r.wait()
                add(self.f_my, H3, CX[self.i])

            def p2_start(self):
                self.r = copy(out_ref.at[pl.ds(self.q_oth, Q3), :],
                              CY.at[self.i], self.sb + P2S, self.p2p)
                self.r.start()

            def p2_fin(self):
                self.r.wait()
                add(self.q_my, Q3, CY[self.i])

            def rs_start(self, s):
                if s == 2:
                    pl.semaphore_wait(self.cred, 1)
                rows = self.q_my + self.rs_send(s) * C3
                self.r = copy(out_ref.at[pl.ds(rows, C3), :],
                              CZ.at[self.i, s % 2], self.sb + RSZS + s,
                              self.snbr)
                self.r.start()

            def rs_fin(self, s):
                self.r.wait()
                add(self.q_my + self.rs_recv(s) * C3, C3, CZ[self.i, s % 2])
                if s == 0:
                    pl.semaphore_signal(self.cred, inc=1,
                                        device_id=self.rnbr,
                                        device_id_type=MESH)

            def ag_start(self, s):
                rows = self.q_my + self.ag_send(s) * C3
                self.r = copy(out_ref.at[pl.ds(rows, C3), :],
                              out_ref.at[pl.ds(rows, C3), :],
                              self.sb + AGZS + s, self.snbr)
                self.r.start()

            def ag_fin(self, s):
                self.r.wait()

            def agq_start(self):
                self.r = copy(out_ref.at[pl.ds(self.q_my, Q3), :],
                              out_ref.at[pl.ds(self.q_my, Q3), :],
                              self.sb + AGQS, self.p2p)
                self.r.start()

            def agq_fin(self):
                self.r.wait()

            def agh_start(self):
                self.r = copy(out_ref.at[pl.ds(self.f_my, H3), :],
                              out_ref.at[pl.ds(self.f_my, H3), :],
                              self.sb + AGHS, self.p1p)
                self.r.start()

            def agh_fin(self):
                self.r.wait()

        chains = [
            Chain(0, 0, True, 1),
            Chain(1, S3, True, 1),
            Chain(2, 2 * S3, False, -1),
            Chain(3, 3 * S3, False, -1),
        ]

        barrier = pltpu.get_barrier_semaphore()
        for nbr in (xp, yp, zl, zr):
            pl.semaphore_signal(barrier, inc=1, device_id=nbr,
                                device_id_type=MESH)
        pl.semaphore_wait(barrier, 4)

        for c in chains:
            mm(c.f_oth, H3)
        for c in chains:
            c.p1_start()
        for c in chains:
            mm(c.f_my, H3)

        for c in chains:
            c.p1_fin()
            c.p2_start()
        for c in chains:
            c.p2_fin()
            c.rs_start(0)
        for s in range(3):
            for c in chains:
                c.rs_fin(s)
                if s < 2:
                    c.rs_start(s + 1)
                else:
                    c.ag_start(0)
        for s in range(3):
            for c in chains:
                c.ag_fin(s)
                if s < 2:
                    c.ag_start(s + 1)
                else:
                    c.agq_start()
        for c in chains:
            c.agq_fin()
            c.agh_start()
        for c in chains:
            c.agh_fin()

        @functools.partial(pl.run_scoped, sem2=pltpu.SemaphoreType.REGULAR)
        def _(sem2):
            for nbr in (xp, yp, zl, zr):
                pl.semaphore_signal(sem2, inc=1, device_id=nbr,
                                    device_id_type=MESH)
            pl.semaphore_wait(sem2, 4)

    return pl.pallas_call(
        body,
        out_shape=jax.ShapeDtypeStruct((M, M), jnp.float32),
        in_specs=[
            pl.BlockSpec(memory_space=pltpu.VMEM),
            pl.BlockSpec(memory_space=pltpu.VMEM),
        ],
        out_specs=pl.BlockSpec(memory_space=pltpu.VMEM),
        scratch_shapes=[
            pltpu.VMEM((4, H3, M), jnp.float32),
            pltpu.VMEM((4, Q3, M), jnp.float32),
            pltpu.VMEM((4, 2, C3, M), jnp.float32),
            pltpu.SemaphoreType.DMA((NSEM,)),
            pltpu.SemaphoreType.DMA((NSEM,)),
            pltpu.SemaphoreType.REGULAR,
            pltpu.SemaphoreType.REGULAR,
            pltpu.SemaphoreType.REGULAR,
            pltpu.SemaphoreType.REGULAR,
        ],
        compiler_params=pltpu.CompilerParams(
            collective_id=0,
            vmem_limit_bytes=60 * 1024 * 1024,
        ),
    )(dy_c, w_c)
